# baseline (speedup 1.0000x reference)
import functools

import jax
import jax.numpy as jnp
from jax.experimental import pallas as pl
from jax.experimental.pallas import tpu as pltpu
from jax.experimental.pallas import tpu_sc as plsc

_EPS = 1e-5
_NC = 2
_NS = 16
_NW = _NC * _NS
_CH = 128
_PH = 16


def _pad_rows(n):
    block = _NS * _CH
    return ((n + 1 + block - 1) // block) * block


def _sc_mesh():
    return plsc.VectorSubcoreMesh(core_axis_name="c", subcore_axis_name="s",
                                  num_cores=_NC, num_subcores=_NS)


def _sc_degree(dst3, npad):
    nch = dst3.shape[1]
    rpt = npad // _NS
    nzc = rpt // _CH

    @functools.partial(
        pl.kernel,
        out_type=jax.ShapeDtypeStruct((_NC * npad, 16), jnp.float32),
        mesh=_sc_mesh(),
        scratch_types=[
            pltpu.VMEM((nch, _CH), jnp.int32),
            pltpu.VMEM((_CH, 16), jnp.float32),
            pltpu.VMEM_SHARED((npad, 16), jnp.float32),
        ],
    )
    def deg_kernel(dst_hbm, out_hbm, dst_v, buf_v, acc_sh):
        cid = jax.lax.axis_index("c")
        sid = jax.lax.axis_index("s")
        wid = sid * _NC + cid

        @pl.loop(0, _CH)
        def _(i):
            buf_v[i, :] = jnp.zeros((16,), jnp.float32)

        @pl.loop(0, nzc)
        def _(z):
            pltpu.sync_copy(buf_v, acc_sh.at[pl.ds(sid * rpt + z * _CH, _CH)])

        pltpu.sync_copy(dst_hbm.at[wid], dst_v)

        @pl.loop(0, _CH)
        def _(i):
            buf_v[i, :] = jnp.ones((16,), jnp.float32)

        plsc.subcore_barrier()

        @pl.loop(0, nch)
        def _(j):
            pltpu.sync_copy(buf_v, acc_sh.at[dst_v.at[j]], add=True)

        plsc.subcore_barrier()

        @pl.loop(0, nzc)
        def _(z):
            base = sid * rpt + z * _CH
            pltpu.sync_copy(acc_sh.at[pl.ds(base, _CH)],
                            out_hbm.at[pl.ds(cid * npad + base, _CH)])

    return deg_kernel(dst3)


def _sc_scatter_rows(table, src3, dst3, npad):
    d = table.shape[1]
    nch = src3.shape[1]
    rpt = npad // _NS
    nzc = rpt // _CH

    nph = nch // _PH

    @functools.partial(
        pl.kernel,
        out_type=jax.ShapeDtypeStruct((_NC * npad, d), jnp.float32),
        mesh=_sc_mesh(),
        scratch_types=[
            pltpu.VMEM((_PH, _CH), jnp.int32),
            pltpu.VMEM((_PH, _CH), jnp.int32),
            pltpu.VMEM((_CH, d), jnp.float32),
            pltpu.VMEM((_CH, d), jnp.float32),
            pltpu.VMEM_SHARED((npad, d), jnp.float32),
            pltpu.SemaphoreType.DMA,
            pltpu.SemaphoreType.DMA,
        ],
    )
    def msg_kernel(table_hbm, src_hbm, dst_hbm, out_hbm,
                   src_v, dst_v, rows_a, rows_b, acc_sh, sem_a, sem_b):
        cid = jax.lax.axis_index("c")
        sid = jax.lax.axis_index("s")
        wid = sid * _NC + cid

        @pl.loop(0, _CH)
        def _(i):
            @pl.loop(0, d, step=16)
            def _(c):
                rows_a[i, pl.ds(c, 16)] = jnp.zeros((16,), jnp.float32)

        @pl.loop(0, nzc)
        def _(z):
            pltpu.sync_copy(rows_a, acc_sh.at[pl.ds(sid * rpt + z * _CH, _CH)])

        plsc.subcore_barrier()

        @pl.loop(0, nph)
        def _(p):
            pltpu.sync_copy(src_hbm.at[wid * nph + p], src_v)
            pltpu.sync_copy(dst_hbm.at[wid * nph + p], dst_v)
            pltpu.async_copy(table_hbm.at[src_v.at[0]], rows_a, sem_a)

            @pl.loop(0, _PH, step=2)
            def _(j):
                pltpu.async_copy(table_hbm.at[src_v.at[j + 1]], rows_b, sem_b)
                pltpu.make_async_copy(table_hbm.at[src_v.at[j]], rows_a,
                                      sem_a).wait()
                pltpu.sync_copy(rows_a, acc_sh.at[dst_v.at[j]], add=True)

                @pl.when(j + 2 < _PH)
                def _():
                    pltpu.async_copy(table_hbm.at[src_v.at[j + 2]], rows_a,
                                     sem_a)

                pltpu.make_async_copy(table_hbm.at[src_v.at[j + 1]], rows_b,
                                      sem_b).wait()
                pltpu.sync_copy(rows_b, acc_sh.at[dst_v.at[j + 1]], add=True)

        plsc.subcore_barrier()

        @pl.loop(0, nzc)
        def _(z):
            base = sid * rpt + z * _CH
            pltpu.sync_copy(acc_sh.at[pl.ds(base, _CH)],
                            out_hbm.at[pl.ds(cid * npad + base, _CH)])

    return msg_kernel(table,
                      src3.reshape(_NW * nph, _PH, _CH),
                      dst3.reshape(_NW * nph, _PH, _CH))


def _tc_matmul(x, w):
    def body(x_ref, w_ref, o_ref):
        o_ref[...] = jnp.dot(x_ref[...], w_ref[...],
                             preferred_element_type=jnp.float32)

    return pl.pallas_call(
        body,
        out_shape=jax.ShapeDtypeStruct((x.shape[0], w.shape[1]), jnp.float32),
    )(x, w)


def _tc_scale(h, degs, npad):
    n, d = h.shape

    def body(h_ref, dp_ref, o_ref):
        deg = dp_ref[pl.ds(0, n), 0:1] + dp_ref[pl.ds(npad, n), 0:1] + 1.0
        dinv = jax.lax.rsqrt(deg)
        o_ref[...] = h_ref[...] * dinv

    return pl.pallas_call(
        body,
        out_shape=jax.ShapeDtypeStruct((n, d), jnp.float32),
    )(h, degs)


def _tc_mid(s, hp, degs, b, g, be, w2, npad):
    n, d = hp.shape

    def body(s_ref, hp_ref, dp_ref, b_ref, g_ref, be_ref, w_ref, o_ref):
        deg = dp_ref[pl.ds(0, n), 0:1] + dp_ref[pl.ds(npad, n), 0:1] + 1.0
        dinv = jax.lax.rsqrt(deg)
        u = (s_ref[pl.ds(0, n), :] + s_ref[pl.ds(npad, n), :]
             + hp_ref[...]) * dinv + b_ref[...]
        mean = jnp.mean(u, axis=0, keepdims=True)
        var = jnp.mean((u - mean) ** 2, axis=0, keepdims=True)
        xh = (u - mean) * jax.lax.rsqrt(var + _EPS)
        t = jnp.maximum(g_ref[...] * xh + be_ref[...], 0.0)
        o_ref[...] = jnp.dot(t, w_ref[...],
                             preferred_element_type=jnp.float32) * dinv

    return pl.pallas_call(
        body,
        out_shape=jax.ShapeDtypeStruct((n, d), jnp.float32),
    )(s, hp, degs, b, g, be, w2)


def _tc_final(s, hp, degs, b, g, be, npad):
    n, d = hp.shape

    def body(s_ref, hp_ref, dp_ref, b_ref, g_ref, be_ref, o_ref):
        deg = dp_ref[pl.ds(0, n), 0:1] + dp_ref[pl.ds(npad, n), 0:1] + 1.0
        dinv = jax.lax.rsqrt(deg)
        u = (s_ref[pl.ds(0, n), :] + s_ref[pl.ds(npad, n), :]
             + hp_ref[...]) * dinv + b_ref[...]
        mean = jnp.mean(u, axis=0, keepdims=True)
        var = jnp.mean((u - mean) ** 2, axis=0, keepdims=True)
        xh = (u - mean) * jax.lax.rsqrt(var + _EPS)
        o_ref[...] = g_ref[...] * xh + be_ref[...]

    return pl.pallas_call(
        body,
        out_shape=jax.ShapeDtypeStruct((n, d), jnp.float32),
    )(s, hp, degs, b, g, be)


def kernel(x, edge_index, W1, b1, g1, be1, W2, b2, g2, be2):
    n, d = x.shape
    e = edge_index.shape[1]
    ei = edge_index.astype(jnp.int32)
    npad = _pad_rows(n)
    nch = -(-e // (_NW * _CH * _PH)) * _PH
    pad = _NW * nch * _CH - e
    src3 = jnp.concatenate(
        [ei[0], jnp.zeros((pad,), jnp.int32)]).reshape(_NW, nch, _CH)
    dst3 = jnp.concatenate(
        [ei[1], jnp.full((pad,), n, jnp.int32)]).reshape(_NW, nch, _CH)

    degs = _sc_degree(dst3, npad)
    h1 = _tc_matmul(x, W1)
    h1p = _tc_scale(h1, degs, npad)
    s1 = _sc_scatter_rows(h1p, src3, dst3, npad)
    h2p = _tc_mid(s1, h1p, degs, b1.reshape(1, d), g1.reshape(1, d),
                  be1.reshape(1, d), W2, npad)
    s2 = _sc_scatter_rows(h2p, src3, dst3, npad)
    return _tc_final(s2, h2p, degs, b2.reshape(1, d), g2.reshape(1, d),
                     be2.reshape(1, d), npad)


# speedup vs baseline: 1.3649x; 1.3649x over previous
import functools

import jax
import jax.numpy as jnp
from jax.experimental import pallas as pl
from jax.experimental.pallas import tpu as pltpu
from jax.experimental.pallas import tpu_sc as plsc

_EPS = 1e-5
_NC = 2
_NS = 16
_NW = _NC * _NS
_CH = 128


def _pad_rows(n):
    block = _NS * _CH
    return ((n + 1 + block - 1) // block) * block


def _sc_mesh():
    return plsc.VectorSubcoreMesh(core_axis_name="c", subcore_axis_name="s",
                                  num_cores=_NC, num_subcores=_NS)


def _sc_degree(dst3, npad):
    nch = dst3.shape[1]
    rpt = npad // _NS
    nzc = rpt // _CH

    @functools.partial(
        pl.kernel,
        out_type=jax.ShapeDtypeStruct((_NC * npad, 16), jnp.float32),
        mesh=_sc_mesh(),
        scratch_types=[
            pltpu.VMEM((nch, _CH), jnp.int32),
            pltpu.VMEM((_CH, 16), jnp.float32),
            pltpu.VMEM_SHARED((npad, 16), jnp.float32),
        ],
    )
    def deg_kernel(dst_hbm, out_hbm, dst_v, buf_v, acc_sh):
        cid = jax.lax.axis_index("c")
        sid = jax.lax.axis_index("s")
        wid = sid * _NC + cid

        @pl.loop(0, _CH)
        def _(i):
            buf_v[i, :] = jnp.zeros((16,), jnp.float32)

        @pl.loop(0, nzc)
        def _(z):
            pltpu.sync_copy(buf_v, acc_sh.at[pl.ds(sid * rpt + z * _CH, _CH)])

        pltpu.sync_copy(dst_hbm.at[wid], dst_v)

        @pl.loop(0, _CH)
        def _(i):
            buf_v[i, :] = jnp.ones((16,), jnp.float32)

        plsc.subcore_barrier()

        @pl.loop(0, nch)
        def _(j):
            pltpu.sync_copy(buf_v, acc_sh.at[dst_v.at[j]], add=True)

        plsc.subcore_barrier()

        @pl.loop(0, nzc)
        def _(z):
            base = sid * rpt + z * _CH
            pltpu.sync_copy(acc_sh.at[pl.ds(base, _CH)],
                            out_hbm.at[pl.ds(cid * npad + base, _CH)])

    return deg_kernel(dst3)


def _sc_scatter_rows(table, src3, dst3, npad):
    d = table.shape[1]
    nch = src3.shape[1]
    rpt = npad // _NS
    nzc = rpt // _CH

    @functools.partial(
        pl.kernel,
        out_type=jax.ShapeDtypeStruct((_NC * npad, d), jnp.float32),
        mesh=_sc_mesh(),
        scratch_types=[
            pltpu.VMEM((nch, _CH), jnp.int32),
            pltpu.VMEM((nch, _CH), jnp.int32),
            pltpu.VMEM((_CH, d), jnp.float32),
            pltpu.VMEM_SHARED((npad, d), jnp.float32),
        ],
    )
    def msg_kernel(table_hbm, src_hbm, dst_hbm, out_hbm,
                   src_v, dst_v, rows_v, acc_sh):
        cid = jax.lax.axis_index("c")
        sid = jax.lax.axis_index("s")
        wid = sid * _NC + cid

        @pl.loop(0, _CH)
        def _(i):
            @pl.loop(0, d, step=16)
            def _(c):
                rows_v[i, pl.ds(c, 16)] = jnp.zeros((16,), jnp.float32)

        @pl.loop(0, nzc)
        def _(z):
            pltpu.sync_copy(rows_v, acc_sh.at[pl.ds(sid * rpt + z * _CH, _CH)])

        pltpu.sync_copy(src_hbm.at[wid], src_v)
        pltpu.sync_copy(dst_hbm.at[wid], dst_v)
        plsc.subcore_barrier()

        @pl.loop(0, nch)
        def _(j):
            pltpu.sync_copy(table_hbm.at[src_v.at[j]], rows_v)
            pltpu.sync_copy(rows_v, acc_sh.at[dst_v.at[j]], add=True)

        plsc.subcore_barrier()

        @pl.loop(0, nzc)
        def _(z):
            base = sid * rpt + z * _CH
            pltpu.sync_copy(acc_sh.at[pl.ds(base, _CH)],
                            out_hbm.at[pl.ds(cid * npad + base, _CH)])

    return msg_kernel(table, src3, dst3)


def _tc_matmul(x, w):
    def body(x_ref, w_ref, o_ref):
        o_ref[...] = jnp.dot(x_ref[...], w_ref[...],
                             preferred_element_type=jnp.float32)

    return pl.pallas_call(
        body,
        out_shape=jax.ShapeDtypeStruct((x.shape[0], w.shape[1]), jnp.float32),
    )(x, w)


def _tc_scale(h, degs, npad):
    n, d = h.shape

    def body(h_ref, dp_ref, o_ref):
        deg = dp_ref[pl.ds(0, n), 0:1] + dp_ref[pl.ds(npad, n), 0:1] + 1.0
        dinv = jax.lax.rsqrt(deg)
        o_ref[...] = h_ref[...] * dinv

    return pl.pallas_call(
        body,
        out_shape=jax.ShapeDtypeStruct((n, d), jnp.float32),
    )(h, degs)


def _tc_mid(s, hp, degs, b, g, be, w2, npad):
    n, d = hp.shape

    def body(s_ref, hp_ref, dp_ref, b_ref, g_ref, be_ref, w_ref, o_ref):
        deg = dp_ref[pl.ds(0, n), 0:1] + dp_ref[pl.ds(npad, n), 0:1] + 1.0
        dinv = jax.lax.rsqrt(deg)
        u = (s_ref[pl.ds(0, n), :] + s_ref[pl.ds(npad, n), :]
             + hp_ref[...]) * dinv + b_ref[...]
        mean = jnp.mean(u, axis=0, keepdims=True)
        var = jnp.mean((u - mean) ** 2, axis=0, keepdims=True)
        xh = (u - mean) * jax.lax.rsqrt(var + _EPS)
        t = jnp.maximum(g_ref[...] * xh + be_ref[...], 0.0)
        o_ref[...] = jnp.dot(t, w_ref[...],
                             preferred_element_type=jnp.float32) * dinv

    return pl.pallas_call(
        body,
        out_shape=jax.ShapeDtypeStruct((n, d), jnp.float32),
    )(s, hp, degs, b, g, be, w2)


def _tc_final(s, hp, degs, b, g, be, npad):
    n, d = hp.shape

    def body(s_ref, hp_ref, dp_ref, b_ref, g_ref, be_ref, o_ref):
        deg = dp_ref[pl.ds(0, n), 0:1] + dp_ref[pl.ds(npad, n), 0:1] + 1.0
        dinv = jax.lax.rsqrt(deg)
        u = (s_ref[pl.ds(0, n), :] + s_ref[pl.ds(npad, n), :]
             + hp_ref[...]) * dinv + b_ref[...]
        mean = jnp.mean(u, axis=0, keepdims=True)
        var = jnp.mean((u - mean) ** 2, axis=0, keepdims=True)
        xh = (u - mean) * jax.lax.rsqrt(var + _EPS)
        o_ref[...] = g_ref[...] * xh + be_ref[...]

    return pl.pallas_call(
        body,
        out_shape=jax.ShapeDtypeStruct((n, d), jnp.float32),
    )(s, hp, degs, b, g, be)


def kernel(x, edge_index, W1, b1, g1, be1, W2, b2, g2, be2):
    n, d = x.shape
    e = edge_index.shape[1]
    ei = edge_index.astype(jnp.int32)
    npad = _pad_rows(n)
    nch = -(-e // (_NW * _CH))
    pad = _NW * nch * _CH - e
    src3 = jnp.concatenate(
        [ei[0], jnp.zeros((pad,), jnp.int32)]).reshape(_NW, nch, _CH)
    dst3 = jnp.concatenate(
        [ei[1], jnp.full((pad,), n, jnp.int32)]).reshape(_NW, nch, _CH)

    degs = _sc_degree(dst3, npad)
    h1 = _tc_matmul(x, W1)
    h1p = _tc_scale(h1, degs, npad)
    s1 = _sc_scatter_rows(h1p, src3, dst3, npad)
    h2p = _tc_mid(s1, h1p, degs, b1.reshape(1, d), g1.reshape(1, d),
                  be1.reshape(1, d), W2, npad)
    s2 = _sc_scatter_rows(h2p, src3, dst3, npad)
    return _tc_final(s2, h2p, degs, b2.reshape(1, d), g2.reshape(1, d),
                     be2.reshape(1, d), npad)


# speedup vs baseline: 2.3554x; 1.7257x over previous
import functools

import jax
import jax.numpy as jnp
from jax.experimental import pallas as pl
from jax.experimental.pallas import tpu as pltpu
from jax.experimental.pallas import tpu_sc as plsc

_EPS = 1e-5
_NC = 2
_NS = 16
_NW = _NC * _NS
_CH = 128


def _pad_rows(n):
    block = _NS * _CH
    return ((n + 1 + block - 1) // block) * block


def _sc_mesh():
    return plsc.VectorSubcoreMesh(core_axis_name="c", subcore_axis_name="s",
                                  num_cores=_NC, num_subcores=_NS)


def _sc_degree(dst3, npad):
    nch = dst3.shape[1]
    rpt = npad // _NS
    nzc = rpt // _CH

    @functools.partial(
        pl.kernel,
        out_type=jax.ShapeDtypeStruct((_NC * npad, 16), jnp.float32),
        mesh=_sc_mesh(),
        scratch_types=[
            pltpu.VMEM((nch, _CH), jnp.int32),
            pltpu.VMEM((_CH, 16), jnp.float32),
            pltpu.VMEM_SHARED((npad, 16), jnp.float32),
        ],
    )
    def deg_kernel(dst_hbm, out_hbm, dst_v, buf_v, acc_sh):
        cid = jax.lax.axis_index("c")
        sid = jax.lax.axis_index("s")
        wid = sid * _NC + cid

        @pl.loop(0, _CH)
        def _(i):
            buf_v[i, :] = jnp.zeros((16,), jnp.float32)

        @pl.loop(0, nzc)
        def _(z):
            pltpu.sync_copy(buf_v, acc_sh.at[pl.ds(sid * rpt + z * _CH, _CH)])

        pltpu.sync_copy(dst_hbm.at[wid], dst_v)

        @pl.loop(0, _CH)
        def _(i):
            buf_v[i, :] = jnp.ones((16,), jnp.float32)

        plsc.subcore_barrier()

        @pl.loop(0, nch)
        def _(j):
            pltpu.sync_copy(buf_v, acc_sh.at[dst_v.at[j]], add=True)

        plsc.subcore_barrier()

        @pl.loop(0, nzc)
        def _(z):
            base = sid * rpt + z * _CH
            pltpu.sync_copy(acc_sh.at[pl.ds(base, _CH)],
                            out_hbm.at[pl.ds(cid * npad + base, _CH)])

    return deg_kernel(dst3)


def _sc_scatter_rows(table, src3, dst3, npad):
    d = table.shape[1]
    nch = src3.shape[1]
    rpt = npad // _NS
    nzc = rpt // _CH

    @functools.partial(
        pl.kernel,
        out_type=jax.ShapeDtypeStruct((_NC * npad, d), jnp.float32),
        mesh=_sc_mesh(),
        scratch_types=[
            pltpu.VMEM((nch, _CH), jnp.int32),
            pltpu.VMEM((nch, _CH), jnp.int32),
            pltpu.VMEM((_CH, d), jnp.float32),
            pltpu.VMEM_SHARED((npad, d), jnp.float32),
        ],
    )
    def msg_kernel(table_hbm, src_hbm, dst_hbm, out_hbm,
                   src_v, dst_v, rows_v, acc_sh):
        cid = jax.lax.axis_index("c")
        sid = jax.lax.axis_index("s")
        wid = sid * _NC + cid

        @pl.loop(0, _CH)
        def _(i):
            @pl.loop(0, d, step=16)
            def _(c):
                rows_v[i, pl.ds(c, 16)] = jnp.zeros((16,), jnp.float32)

        @pl.loop(0, nzc)
        def _(z):
            pltpu.sync_copy(rows_v, acc_sh.at[pl.ds(sid * rpt + z * _CH, _CH)])

        pltpu.sync_copy(src_hbm.at[wid], src_v)
        pltpu.sync_copy(dst_hbm.at[wid], dst_v)
        plsc.subcore_barrier()

        @pl.loop(0, nch)
        def _(j):
            pltpu.sync_copy(table_hbm.at[src_v.at[j]], rows_v)
            pltpu.sync_copy(rows_v, acc_sh.at[dst_v.at[j]], add=True)

        plsc.subcore_barrier()

        @pl.loop(0, nzc)
        def _(z):
            base = sid * rpt + z * _CH
            pltpu.sync_copy(acc_sh.at[pl.ds(base, _CH)],
                            out_hbm.at[pl.ds(cid * npad + base, _CH)])

    return msg_kernel(table, src3, dst3)


def _tc_matmul(x, w):
    def body(x_ref, w_ref, o_ref):
        o_ref[...] = jnp.dot(x_ref[...], w_ref[...],
                             preferred_element_type=jnp.float32)

    return pl.pallas_call(
        body,
        out_shape=jax.ShapeDtypeStruct((x.shape[0], w.shape[1]), jnp.float32),
    )(x, w)


def _tc_scale(h, degs, npad):
    n, d = h.shape

    def body(h_ref, dp_ref, o_ref):
        deg = dp_ref[pl.ds(0, n), 0:1] + dp_ref[pl.ds(npad, n), 0:1] + 1.0
        dinv = jax.lax.rsqrt(deg)
        o_ref[...] = h_ref[...] * dinv

    return pl.pallas_call(
        body,
        out_shape=jax.ShapeDtypeStruct((n, d), jnp.float32),
    )(h, degs)


def _tc_mid(s, hp, degs, b, g, be, w2, npad):
    n, d = hp.shape

    def body(s_ref, hp_ref, dp_ref, b_ref, g_ref, be_ref, w_ref, o_ref):
        deg = dp_ref[pl.ds(0, n), 0:1] + dp_ref[pl.ds(npad, n), 0:1] + 1.0
        dinv = jax.lax.rsqrt(deg)
        u = (s_ref[pl.ds(0, n), :] + s_ref[pl.ds(npad, n), :]
             + hp_ref[...]) * dinv + b_ref[...]
        mean = jnp.mean(u, axis=0, keepdims=True)
        var = jnp.mean((u - mean) ** 2, axis=0, keepdims=True)
        xh = (u - mean) * jax.lax.rsqrt(var + _EPS)
        t = jnp.maximum(g_ref[...] * xh + be_ref[...], 0.0)
        o_ref[...] = jnp.dot(t, w_ref[...],
                             preferred_element_type=jnp.float32) * dinv

    return pl.pallas_call(
        body,
        out_shape=jax.ShapeDtypeStruct((n, d), jnp.float32),
    )(s, hp, degs, b, g, be, w2)


def _tc_final(s, hp, degs, b, g, be, npad):
    n, d = hp.shape

    def body(s_ref, hp_ref, dp_ref, b_ref, g_ref, be_ref, o_ref):
        deg = dp_ref[pl.ds(0, n), 0:1] + dp_ref[pl.ds(npad, n), 0:1] + 1.0
        dinv = jax.lax.rsqrt(deg)
        u = (s_ref[pl.ds(0, n), :] + s_ref[pl.ds(npad, n), :]
             + hp_ref[...]) * dinv + b_ref[...]
        mean = jnp.mean(u, axis=0, keepdims=True)
        var = jnp.mean((u - mean) ** 2, axis=0, keepdims=True)
        xh = (u - mean) * jax.lax.rsqrt(var + _EPS)
        o_ref[...] = g_ref[...] * xh + be_ref[...]

    return pl.pallas_call(
        body,
        out_shape=jax.ShapeDtypeStruct((n, d), jnp.float32),
    )(s, hp, degs, b, g, be)


def kernel(x, edge_index, W1, b1, g1, be1, W2, b2, g2, be2):
    n, d = x.shape
    e = edge_index.shape[1]
    ei = edge_index.astype(jnp.int32)
    npad = _pad_rows(n)
    nch = -(-e // (_NW * _CH))
    pad = _NW * nch * _CH - e
    pidx = jnp.arange(pad, dtype=jnp.int32)
    src3 = jnp.concatenate(
        [ei[0], pidx % jnp.int32(n)]).reshape(_NW, nch, _CH)
    dst3 = jnp.concatenate(
        [ei[1], n + pidx % jnp.int32(npad - n)]).reshape(_NW, nch, _CH)

    degs = _sc_degree(dst3, npad)
    h1 = _tc_matmul(x, W1)
    h1p = _tc_scale(h1, degs, npad)
    s1 = _sc_scatter_rows(h1p, src3, dst3, npad)
    h2p = _tc_mid(s1, h1p, degs, b1.reshape(1, d), g1.reshape(1, d),
                  be1.reshape(1, d), W2, npad)
    s2 = _sc_scatter_rows(h2p, src3, dst3, npad)
    return _tc_final(s2, h2p, degs, b2.reshape(1, d), g2.reshape(1, d),
                     be2.reshape(1, d), npad)


# speedup vs baseline: 3.3389x; 1.4175x over previous
import functools

import jax
import jax.numpy as jnp
from jax.experimental import pallas as pl
from jax.experimental.pallas import tpu as pltpu
from jax.experimental.pallas import tpu_sc as plsc

_EPS = 1e-5
_NC = 2
_NS = 16
_NW = _NC * _NS
_CH = 128
_WB = 128


def _pad_rows(n):
    block = _NS * _WB
    return ((n + 1 + block - 1) // block) * block


def _sc_mesh():
    return plsc.VectorSubcoreMesh(core_axis_name="c", subcore_axis_name="s",
                                  num_cores=_NC, num_subcores=_NS)


def _sc_degree(dst3, npad):
    nch = dst3.shape[1]
    rpt = npad // _NS
    nzc = rpt // _WB

    @functools.partial(
        pl.kernel,
        out_type=jax.ShapeDtypeStruct((_NC * npad, 16), jnp.float32),
        mesh=_sc_mesh(),
        scratch_types=[
            pltpu.VMEM((nch, _CH), jnp.int32),
            pltpu.VMEM((_CH, 16), jnp.float32),
            pltpu.VMEM_SHARED((npad, 16), jnp.float32),
        ],
    )
    def deg_kernel(dst_hbm, out_hbm, dst_v, buf_v, acc_sh):
        cid = jax.lax.axis_index("c")
        sid = jax.lax.axis_index("s")
        wid = sid * _NC + cid

        @pl.loop(0, _CH)
        def _(i):
            buf_v[i, :] = jnp.zeros((16,), jnp.float32)

        @pl.loop(0, nzc)
        def _(z):
            pltpu.sync_copy(buf_v, acc_sh.at[pl.ds(sid * rpt + z * _WB, _WB)])

        pltpu.sync_copy(dst_hbm.at[wid], dst_v)

        @pl.loop(0, _CH)
        def _(i):
            buf_v[i, :] = jnp.ones((16,), jnp.float32)

        plsc.subcore_barrier()

        @pl.loop(0, nch)
        def _(j):
            pltpu.sync_copy(buf_v, acc_sh.at[dst_v.at[j]], add=True)

        plsc.subcore_barrier()

        @pl.loop(0, nzc)
        def _(z):
            base = sid * rpt + z * _WB
            pltpu.sync_copy(acc_sh.at[pl.ds(base, _WB)],
                            out_hbm.at[pl.ds(cid * npad + base, _WB)])

    return deg_kernel(dst3)


def _sc_scatter_rows(table, src3, dst3, npad):
    d = table.shape[1]
    nch = src3.shape[1]
    rpt = npad // _NS
    nzc = rpt // _WB

    @functools.partial(
        pl.kernel,
        out_type=jax.ShapeDtypeStruct((_NC * npad, d), jnp.float32),
        mesh=_sc_mesh(),
        scratch_types=[
            pltpu.VMEM((nch, _CH), jnp.int32),
            pltpu.VMEM((nch, _CH), jnp.int32),
            pltpu.VMEM((_CH, d), jnp.float32),
            pltpu.VMEM((_CH, d), jnp.float32),
            pltpu.VMEM_SHARED((npad, d), jnp.float32),
            pltpu.SemaphoreType.DMA,
            pltpu.SemaphoreType.DMA,
        ],
    )
    def msg_kernel(table_hbm, src_hbm, dst_hbm, out_hbm,
                   src_v, dst_v, rows_a, rows_b, acc_sh, sem_a, sem_b):
        cid = jax.lax.axis_index("c")
        sid = jax.lax.axis_index("s")
        wid = sid * _NC + cid

        @pl.loop(0, _CH)
        def _(i):
            @pl.loop(0, d, step=16)
            def _(c):
                rows_a[i, pl.ds(c, 16)] = jnp.zeros((16,), jnp.float32)

        @pl.loop(0, nzc)
        def _(z):
            pltpu.sync_copy(rows_a,
                            acc_sh.at[pl.ds(sid * rpt + z * _WB, _WB)])

        plsc.subcore_barrier()

        rows = (rows_a, rows_b)
        sems = (sem_a, sem_b)
        for ph in range(2):
            pltpu.sync_copy(src_hbm.at[wid * 2 + ph], src_v)
            pltpu.sync_copy(dst_hbm.at[wid * 2 + ph], dst_v)
            pend = [None, None]
            pend[0] = pltpu.async_copy(table_hbm.at[src_v.at[0]],
                                       rows_a, sem_a)
            for j in range(nch):
                p = j % 2
                if j + 1 < nch:
                    pend[1 - p] = pltpu.async_copy(
                        table_hbm.at[src_v.at[j + 1]], rows[1 - p],
                        sems[1 - p])
                pend[p].wait()
                pltpu.sync_copy(rows[p], acc_sh.at[dst_v.at[j]], add=True)

        plsc.subcore_barrier()

        @pl.loop(0, nzc)
        def _(z):
            base = sid * rpt + z * _WB
            pltpu.sync_copy(acc_sh.at[pl.ds(base, _WB)],
                            out_hbm.at[pl.ds(cid * npad + base, _WB)])

    return msg_kernel(table, src3, dst3)


def _tc_matmul(x, w):
    def body(x_ref, w_ref, o_ref):
        o_ref[...] = jnp.dot(x_ref[...], w_ref[...],
                             preferred_element_type=jnp.float32)

    return pl.pallas_call(
        body,
        out_shape=jax.ShapeDtypeStruct((x.shape[0], w.shape[1]), jnp.float32),
    )(x, w)


def _tc_scale(h, degs, npad):
    n, d = h.shape

    def body(h_ref, dp_ref, o_ref):
        deg = dp_ref[pl.ds(0, n), 0:1] + dp_ref[pl.ds(npad, n), 0:1] + 1.0
        dinv = jax.lax.rsqrt(deg)
        o_ref[...] = h_ref[...] * dinv

    return pl.pallas_call(
        body,
        out_shape=jax.ShapeDtypeStruct((n, d), jnp.float32),
    )(h, degs)


def _tc_mid(s, hp, degs, b, g, be, w2, npad):
    n, d = hp.shape

    def body(s_ref, hp_ref, dp_ref, b_ref, g_ref, be_ref, w_ref, o_ref):
        deg = dp_ref[pl.ds(0, n), 0:1] + dp_ref[pl.ds(npad, n), 0:1] + 1.0
        dinv = jax.lax.rsqrt(deg)
        u = (s_ref[pl.ds(0, n), :] + s_ref[pl.ds(npad, n), :]
             + hp_ref[...]) * dinv + b_ref[...]
        mean = jnp.mean(u, axis=0, keepdims=True)
        var = jnp.mean((u - mean) ** 2, axis=0, keepdims=True)
        xh = (u - mean) * jax.lax.rsqrt(var + _EPS)
        t = jnp.maximum(g_ref[...] * xh + be_ref[...], 0.0)
        o_ref[...] = jnp.dot(t, w_ref[...],
                             preferred_element_type=jnp.float32) * dinv

    return pl.pallas_call(
        body,
        out_shape=jax.ShapeDtypeStruct((n, d), jnp.float32),
    )(s, hp, degs, b, g, be, w2)


def _tc_final(s, hp, degs, b, g, be, npad):
    n, d = hp.shape

    def body(s_ref, hp_ref, dp_ref, b_ref, g_ref, be_ref, o_ref):
        deg = dp_ref[pl.ds(0, n), 0:1] + dp_ref[pl.ds(npad, n), 0:1] + 1.0
        dinv = jax.lax.rsqrt(deg)
        u = (s_ref[pl.ds(0, n), :] + s_ref[pl.ds(npad, n), :]
             + hp_ref[...]) * dinv + b_ref[...]
        mean = jnp.mean(u, axis=0, keepdims=True)
        var = jnp.mean((u - mean) ** 2, axis=0, keepdims=True)
        xh = (u - mean) * jax.lax.rsqrt(var + _EPS)
        o_ref[...] = g_ref[...] * xh + be_ref[...]

    return pl.pallas_call(
        body,
        out_shape=jax.ShapeDtypeStruct((n, d), jnp.float32),
    )(s, hp, degs, b, g, be)


def kernel(x, edge_index, W1, b1, g1, be1, W2, b2, g2, be2):
    n, d = x.shape
    e = edge_index.shape[1]
    ei = edge_index.astype(jnp.int32)
    npad = _pad_rows(n)
    nch = 2 * -(-e // (_NW * _CH * 2))
    pad = _NW * nch * _CH - e
    pidx = jnp.arange(pad, dtype=jnp.int32)
    src3 = jnp.concatenate(
        [ei[0], pidx % jnp.int32(n)]).reshape(_NW, nch, _CH)
    dst3 = jnp.concatenate(
        [ei[1], n + pidx % jnp.int32(npad - n)]).reshape(_NW, nch, _CH)

    src2 = src3.reshape(_NW * 2, nch // 2, _CH)
    dst2 = dst3.reshape(_NW * 2, nch // 2, _CH)

    degs = _sc_degree(dst3, npad)
    h1 = _tc_matmul(x, W1)
    h1p = _tc_scale(h1, degs, npad)
    s1 = _sc_scatter_rows(h1p, src2, dst2, npad)
    h2p = _tc_mid(s1, h1p, degs, b1.reshape(1, d), g1.reshape(1, d),
                  be1.reshape(1, d), W2, npad)
    s2 = _sc_scatter_rows(h2p, src2, dst2, npad)
    return _tc_final(s2, h2p, degs, b2.reshape(1, d), g2.reshape(1, d),
                     be2.reshape(1, d), npad)


# speedup vs baseline: 3.3782x; 1.0118x over previous
import functools

import jax
import jax.numpy as jnp
from jax.experimental import pallas as pl
from jax.experimental.pallas import tpu as pltpu
from jax.experimental.pallas import tpu_sc as plsc

_EPS = 1e-5
_NC = 2
_NS = 16
_NW = _NC * _NS
_CH = 128
_WB = 128


def _pad_rows(n):
    block = _NS * _WB
    return ((n + 1 + block - 1) // block) * block


def _sc_mesh():
    return plsc.VectorSubcoreMesh(core_axis_name="c", subcore_axis_name="s",
                                  num_cores=_NC, num_subcores=_NS)


def _sc_degree(dst3, npad):
    nch = dst3.shape[1]
    rpt = npad // _NS
    nzc = rpt // _WB

    @functools.partial(
        pl.kernel,
        out_type=jax.ShapeDtypeStruct((_NC * npad, 16), jnp.float32),
        mesh=_sc_mesh(),
        scratch_types=[
            pltpu.VMEM((nch, _CH), jnp.int32),
            pltpu.VMEM((_CH, 16), jnp.float32),
            pltpu.VMEM_SHARED((npad, 16), jnp.float32),
            pltpu.SemaphoreType.DMA,
        ],
    )
    def deg_kernel(dst_hbm, out_hbm, dst_v, buf_v, acc_sh, sem):
        cid = jax.lax.axis_index("c")
        sid = jax.lax.axis_index("s")
        wid = sid * _NC + cid

        @pl.loop(0, _CH)
        def _(i):
            buf_v[i, :] = jnp.zeros((16,), jnp.float32)

        @pl.loop(0, nzc)
        def _(z):
            pltpu.sync_copy(buf_v, acc_sh.at[pl.ds(sid * rpt + z * _WB, _WB)])

        pltpu.sync_copy(dst_hbm.at[wid], dst_v)

        @pl.loop(0, _CH)
        def _(i):
            buf_v[i, :] = jnp.ones((16,), jnp.float32)

        plsc.subcore_barrier()

        pend = []
        for j in range(nch):
            pend.append(pltpu.async_copy(buf_v, acc_sh.at[dst_v.at[j]],
                                         sem, add=True))
        for h in pend:
            h.wait()

        plsc.subcore_barrier()

        @pl.loop(0, nzc)
        def _(z):
            base = sid * rpt + z * _WB
            pltpu.sync_copy(acc_sh.at[pl.ds(base, _WB)],
                            out_hbm.at[pl.ds(cid * npad + base, _WB)])

    return deg_kernel(dst3)


def _sc_scatter_rows(table, src3, dst3, npad):
    d = table.shape[1]
    nch = src3.shape[1]
    rpt = npad // _NS
    nzc = rpt // _WB

    @functools.partial(
        pl.kernel,
        out_type=jax.ShapeDtypeStruct((_NC * npad, d), jnp.float32),
        mesh=_sc_mesh(),
        scratch_types=[
            pltpu.VMEM((nch, _CH), jnp.int32),
            pltpu.VMEM((nch, _CH), jnp.int32),
            pltpu.VMEM((_CH, d), jnp.float32),
            pltpu.VMEM((_CH, d), jnp.float32),
            pltpu.VMEM_SHARED((npad, d), jnp.float32),
            pltpu.SemaphoreType.DMA,
            pltpu.SemaphoreType.DMA,
        ],
    )
    def msg_kernel(table_hbm, src_hbm, dst_hbm, out_hbm,
                   src_v, dst_v, rows_a, rows_b, acc_sh, sem_a, sem_b):
        cid = jax.lax.axis_index("c")
        sid = jax.lax.axis_index("s")
        wid = sid * _NC + cid

        @pl.loop(0, _CH)
        def _(i):
            @pl.loop(0, d, step=16)
            def _(c):
                rows_a[i, pl.ds(c, 16)] = jnp.zeros((16,), jnp.float32)

        @pl.loop(0, nzc)
        def _(z):
            pltpu.sync_copy(rows_a,
                            acc_sh.at[pl.ds(sid * rpt + z * _WB, _WB)])

        plsc.subcore_barrier()

        rows = (rows_a, rows_b)
        sems = (sem_a, sem_b)
        for ph in range(2):
            pltpu.sync_copy(src_hbm.at[wid * 2 + ph], src_v)
            pltpu.sync_copy(dst_hbm.at[wid * 2 + ph], dst_v)
            pend = [None, None]
            pend[0] = pltpu.async_copy(table_hbm.at[src_v.at[0]],
                                       rows_a, sem_a)
            for j in range(nch):
                p = j % 2
                if j + 1 < nch:
                    pend[1 - p] = pltpu.async_copy(
                        table_hbm.at[src_v.at[j + 1]], rows[1 - p],
                        sems[1 - p])
                pend[p].wait()
                pltpu.sync_copy(rows[p], acc_sh.at[dst_v.at[j]], add=True)

        plsc.subcore_barrier()

        @pl.loop(0, nzc)
        def _(z):
            base = sid * rpt + z * _WB
            pltpu.sync_copy(acc_sh.at[pl.ds(base, _WB)],
                            out_hbm.at[pl.ds(cid * npad + base, _WB)])

    return msg_kernel(table, src3, dst3)


def _tc_mm1(x, w, degs, npad):
    n = x.shape[0]

    def body(x_ref, w_ref, dp_ref, o_ref):
        deg = dp_ref[pl.ds(0, n), 0:1] + dp_ref[pl.ds(npad, n), 0:1] + 1.0
        dinv = jax.lax.rsqrt(deg)
        o_ref[...] = jnp.dot(x_ref[...], w_ref[...],
                             preferred_element_type=jnp.float32) * dinv

    return pl.pallas_call(
        body,
        out_shape=jax.ShapeDtypeStruct((n, w.shape[1]), jnp.float32),
    )(x, w, degs)


def _tc_mid(s, hp, degs, b, g, be, w2, npad):
    n, d = hp.shape

    def body(s_ref, hp_ref, dp_ref, b_ref, g_ref, be_ref, w_ref, o_ref):
        deg = dp_ref[pl.ds(0, n), 0:1] + dp_ref[pl.ds(npad, n), 0:1] + 1.0
        dinv = jax.lax.rsqrt(deg)
        u = (s_ref[pl.ds(0, n), :] + s_ref[pl.ds(npad, n), :]
             + hp_ref[...]) * dinv + b_ref[...]
        mean = jnp.mean(u, axis=0, keepdims=True)
        var = jnp.mean((u - mean) ** 2, axis=0, keepdims=True)
        xh = (u - mean) * jax.lax.rsqrt(var + _EPS)
        t = jnp.maximum(g_ref[...] * xh + be_ref[...], 0.0)
        o_ref[...] = jnp.dot(t, w_ref[...],
                             preferred_element_type=jnp.float32) * dinv

    return pl.pallas_call(
        body,
        out_shape=jax.ShapeDtypeStruct((n, d), jnp.float32),
    )(s, hp, degs, b, g, be, w2)


def _tc_final(s, hp, degs, b, g, be, npad):
    n, d = hp.shape

    def body(s_ref, hp_ref, dp_ref, b_ref, g_ref, be_ref, o_ref):
        deg = dp_ref[pl.ds(0, n), 0:1] + dp_ref[pl.ds(npad, n), 0:1] + 1.0
        dinv = jax.lax.rsqrt(deg)
        u = (s_ref[pl.ds(0, n), :] + s_ref[pl.ds(npad, n), :]
             + hp_ref[...]) * dinv + b_ref[...]
        mean = jnp.mean(u, axis=0, keepdims=True)
        var = jnp.mean((u - mean) ** 2, axis=0, keepdims=True)
        xh = (u - mean) * jax.lax.rsqrt(var + _EPS)
        o_ref[...] = g_ref[...] * xh + be_ref[...]

    return pl.pallas_call(
        body,
        out_shape=jax.ShapeDtypeStruct((n, d), jnp.float32),
    )(s, hp, degs, b, g, be)


def kernel(x, edge_index, W1, b1, g1, be1, W2, b2, g2, be2):
    n, d = x.shape
    e = edge_index.shape[1]
    ei = edge_index.astype(jnp.int32)
    npad = _pad_rows(n)
    nch = 2 * -(-e // (_NW * _CH * 2))
    pad = _NW * nch * _CH - e
    pidx = jnp.arange(pad, dtype=jnp.int32)
    src3 = jnp.concatenate(
        [ei[0], pidx % jnp.int32(n)]).reshape(_NW, nch, _CH)
    dst3 = jnp.concatenate(
        [ei[1], n + pidx % jnp.int32(npad - n)]).reshape(_NW, nch, _CH)

    src2 = src3.reshape(_NW * 2, nch // 2, _CH)
    dst2 = dst3.reshape(_NW * 2, nch // 2, _CH)

    degs = _sc_degree(dst3, npad)
    h1p = _tc_mm1(x, W1, degs, npad)
    s1 = _sc_scatter_rows(h1p, src2, dst2, npad)
    h2p = _tc_mid(s1, h1p, degs, b1.reshape(1, d), g1.reshape(1, d),
                  be1.reshape(1, d), W2, npad)
    s2 = _sc_scatter_rows(h2p, src2, dst2, npad)
    return _tc_final(s2, h2p, degs, b2.reshape(1, d), g2.reshape(1, d),
                     be2.reshape(1, d), npad)
